# baseline (speedup 1.0000x reference)
import functools

import jax
import jax.numpy as jnp
from jax.experimental import pallas as pl
from jax.experimental.pallas import tpu as pltpu

T = 2048
H = 1024
F = 4096
E = 8
K = 2

FB = 512
NF = F // FB
TC = 512
NC = T // TC


def _router_body(x_ref, wr_ref, comb_ref):
    logits = jnp.dot(x_ref[...], wr_ref[...], preferred_element_type=jnp.float32)
    eidx = jax.lax.broadcasted_iota(jnp.int32, (T, E), 1)
    m1 = jnp.max(logits, axis=1, keepdims=True)
    i1 = jnp.min(jnp.where(logits == m1, eidx, E), axis=1, keepdims=True)
    l2 = jnp.where(eidx == i1, -jnp.inf, logits)
    m2 = jnp.max(l2, axis=1, keepdims=True)
    i2 = jnp.min(jnp.where(l2 == m2, eidx, E), axis=1, keepdims=True)
    e2 = jnp.exp(m2 - m1)
    p1 = 1.0 / (1.0 + e2)
    p2 = e2 / (1.0 + e2)
    comb_ref[...] = jnp.where(eidx == i1, p1, 0.0) + jnp.where(eidx == i2, p2, 0.0)


def _ffn_body(xb_ref, w1_ref, w2_ref, comb_ref, y_ref, acc_ref, yacc_ref):
    e = pl.program_id(0)
    fb = pl.program_id(1)

    for c in range(NC):
        sl = (pl.ds(TC * c, TC), slice(None))
        xa = xb_ref[sl]
        h = jnp.dot(xa, w1_ref[0], preferred_element_type=jnp.float32)
        a = jax.nn.gelu(h)
        pacc = jnp.dot(a.astype(jnp.bfloat16), w2_ref[0],
                       preferred_element_type=jnp.float32)

        @pl.when(fb == 0)
        def _():
            acc_ref[sl] = pacc

        @pl.when(fb != 0)
        def _():
            acc_ref[sl] = acc_ref[sl] + pacc

    @pl.when(fb == NF - 1)
    def _():
        eidx = jax.lax.broadcasted_iota(jnp.int32, (T, E), 1)
        cw = jnp.sum(jnp.where(eidx == e, comb_ref[...], 0.0), axis=1,
                     keepdims=True)
        contrib = cw * acc_ref[...]

        @pl.when(e == 0)
        def _():
            yacc_ref[...] = contrib

        @pl.when(e != 0)
        def _():
            yacc_ref[...] = yacc_ref[...] + contrib

        @pl.when(e == E - 1)
        def _():
            y_ref[...] = yacc_ref[...]


@functools.partial(jax.jit, static_argnames=("interpret",))
def kernel(x, Wr, W1, W2, interpret=False):
    comb = pl.pallas_call(
        _router_body,
        out_shape=jax.ShapeDtypeStruct((T, E), jnp.float32),
        interpret=interpret,
    )(x, Wr)

    xb = x.astype(jnp.bfloat16)
    w1b = W1.astype(jnp.bfloat16)
    w2b = W2.astype(jnp.bfloat16)

    y = pl.pallas_call(
        _ffn_body,
        grid=(E, NF),
        in_specs=[
            pl.BlockSpec((T, H), lambda e, fb: (0, 0)),
            pl.BlockSpec((1, H, FB), lambda e, fb: (e, 0, fb)),
            pl.BlockSpec((1, FB, H), lambda e, fb: (e, fb, 0)),
            pl.BlockSpec((T, E), lambda e, fb: (0, 0)),
        ],
        out_specs=pl.BlockSpec((T, H), lambda e, fb: (0, 0)),
        out_shape=jax.ShapeDtypeStruct((T, H), jnp.float32),
        scratch_shapes=[
            pltpu.VMEM((T, H), jnp.float32),
            pltpu.VMEM((T, H), jnp.float32),
        ],
        compiler_params=pltpu.CompilerParams(
            dimension_semantics=("arbitrary", "arbitrary"),
        ),
        interpret=interpret,
    )(xb, w1b, w2b, comb)
    return y


# speedup vs baseline: 1.2653x; 1.2653x over previous
import dataclasses
import functools

import jax
import jax.numpy as jnp
from jax import lax
from jax.experimental import pallas as pl
from jax.experimental.pallas import tpu as pltpu
from jax.experimental.pallas import tpu_sc as plsc

T = 2048
H = 1024
F = 4096
E = 8
K = 2

NENT = T * K
CHUNK = 256
NCHUNK = NENT // CHUNK + E
NSLOT = NCHUNK * CHUNK
NCPAD = 32

NWORK = 32
GW = 64


def _router_body(x_ref, wr_ref, pos_ref, prob_ref, expv_ref, valid_ref):
    logits = jnp.dot(x_ref[...], wr_ref[...], preferred_element_type=jnp.float32)
    eidx = lax.broadcasted_iota(jnp.int32, (T, E), 1)
    m1 = jnp.max(logits, axis=1, keepdims=True)
    i1 = jnp.min(jnp.where(logits == m1, eidx, E), axis=1, keepdims=True)
    l2 = jnp.where(eidx == i1, -jnp.inf, logits)
    m2 = jnp.max(l2, axis=1, keepdims=True)
    i2 = jnp.min(jnp.where(l2 == m2, eidx, E), axis=1, keepdims=True)
    e2 = jnp.exp(m2 - m1)
    p1 = 1.0 / (1.0 + e2)
    p2 = e2 / (1.0 + e2)

    a0 = jnp.where(eidx == i1, 1.0, 0.0)
    a1 = jnp.where(eidx == i2, 1.0, 0.0)
    b = a0 + a1

    r_iota = lax.broadcasted_iota(jnp.int32, (T, T), 0)
    c_iota = lax.broadcasted_iota(jnp.int32, (T, T), 1)
    ltri = jnp.where(r_iota > c_iota, 1.0, 0.0).astype(jnp.bfloat16)
    s = jnp.dot(ltri, b.astype(jnp.bfloat16), preferred_element_type=jnp.float32)

    counts = jnp.sum(b, axis=0, keepdims=True)
    nch = jnp.floor((counts + (CHUNK - 1)) * (1.0 / CHUNK))
    ru = lax.broadcasted_iota(jnp.int32, (E, E), 0)
    cu = lax.broadcasted_iota(jnp.int32, (E, E), 1)
    utri = jnp.where(ru < cu, 1.0, 0.0).astype(jnp.bfloat16)
    choff = jnp.dot(nch.astype(jnp.bfloat16), utri,
                    preferred_element_type=jnp.float32)
    slotbase = choff * float(CHUNK)

    base_bc = jnp.broadcast_to(slotbase, (T, E))
    pos0 = jnp.sum(a0 * (base_bc + s), axis=1, keepdims=True)
    pos1 = jnp.sum(a1 * (base_bc + s), axis=1, keepdims=True)

    pos_ref[...] = jnp.concatenate([pos0, pos1], axis=1).astype(jnp.int32)
    prob_ref[...] = jnp.concatenate([p1, p2], axis=1)

    total = jnp.sum(nch, axis=1, keepdims=True)
    cidx = lax.broadcasted_iota(jnp.int32, (NCPAD, E), 0).astype(jnp.float32)
    off_bc = jnp.broadcast_to(choff, (NCPAD, E))
    expv = jnp.sum(jnp.where(cidx >= off_bc, 1.0, 0.0), axis=1,
                   keepdims=True) - 1.0
    expv = jnp.clip(expv, 0.0, float(E - 1))
    expv_ref[...] = expv.astype(jnp.int32)
    cidx1 = lax.broadcasted_iota(jnp.int32, (NCPAD, 1), 0).astype(jnp.float32)
    valid_ref[...] = (cidx1 < jnp.broadcast_to(total, (NCPAD, 1))).astype(jnp.int32)


def _router(x, Wr):
    return pl.pallas_call(
        _router_body,
        out_shape=(
            jax.ShapeDtypeStruct((T, K), jnp.int32),
            jax.ShapeDtypeStruct((T, K), jnp.float32),
            jax.ShapeDtypeStruct((NCPAD, 1), jnp.int32),
            jax.ShapeDtypeStruct((NCPAD, 1), jnp.int32),
        ),
    )(x, Wr)


def _sc_mesh():
    return plsc.VectorSubcoreMesh(core_axis_name="c", subcore_axis_name="s")


def _sc_compiler_params():
    cp = pltpu.CompilerParams()
    if "needs_layout_passes" in pltpu.CompilerParams.__dataclass_fields__:
        cp = dataclasses.replace(cp, needs_layout_passes=False)
    return cp


def _build_gidx(pos_flat, zeros_slot):
    @functools.partial(
        pl.kernel,
        mesh=_sc_mesh(),
        out_type=jax.ShapeDtypeStruct((NSLOT,), jnp.int32),
        compiler_params=_sc_compiler_params(),
        scratch_types=[
            pltpu.VMEM((NSLOT,), jnp.int32),
            pltpu.VMEM((NENT,), jnp.int32),
        ],
    )
    def k(pos_hbm, zeros_hbm, gidx_hbm, gidx_v, pos_v):
        wid = lax.axis_index("s") * 2 + lax.axis_index("c")

        @pl.when(wid == 0)
        def _():
            pltpu.sync_copy(zeros_hbm, gidx_v)
            pltpu.sync_copy(pos_hbm, pos_v)

            @pl.loop(0, NENT // 16)
            def _(i):
                idx = pos_v[pl.ds(i * 16, 16)]
                vals = lax.shift_right_logical(
                    lax.iota(jnp.int32, 16) + i * 16, 1)
                plsc.store_scatter(gidx_v, [idx], vals)

            pltpu.sync_copy(gidx_v, gidx_hbm)

    return k(pos_flat, zeros_slot)


def _sc_gather(data, idx_flat, nrows):
    nwin = nrows // (NWORK * GW)

    @functools.partial(
        pl.kernel,
        mesh=_sc_mesh(),
        out_type=jax.ShapeDtypeStruct((nrows, H), jnp.float32),
        scratch_types=[
            pltpu.VMEM((GW,), jnp.int32),
            pltpu.VMEM((GW, H), jnp.float32),
            pltpu.SemaphoreType.DMA,
        ],
    )
    def k(data_hbm, idx_hbm, out_hbm, idx_v, rows_v, sem):
        wid = lax.axis_index("s") * 2 + lax.axis_index("c")

        @pl.loop(0, nwin)
        def _(j):
            base = (wid * nwin + j) * GW
            pltpu.sync_copy(idx_hbm.at[pl.ds(base, GW)], idx_v)
            pltpu.async_copy(data_hbm.at[idx_v], rows_v, sem).wait()
            pltpu.sync_copy(rows_v, out_hbm.at[pl.ds(base, GW)])

    return k(data, idx_flat)


def _ffn_body(expv_ref, valid_ref, xg_ref, w1_ref, w2_ref, out_ref):
    c = pl.program_id(0)

    @pl.when(valid_ref[c] != 0)
    def _():
        xa = xg_ref[...].astype(jnp.bfloat16)
        h = jnp.dot(xa, w1_ref[0], preferred_element_type=jnp.float32)
        a = jax.nn.gelu(h)
        out_ref[...] = jnp.dot(a.astype(jnp.bfloat16), w2_ref[0],
                               preferred_element_type=jnp.float32)


def _ffn(expv, valid, xg, w1b, w2b):
    grid_spec = pltpu.PrefetchScalarGridSpec(
        num_scalar_prefetch=2,
        grid=(NCHUNK,),
        in_specs=[
            pl.BlockSpec((CHUNK, H), lambda c, expv, valid: (c, 0)),
            pl.BlockSpec((1, H, F), lambda c, expv, valid: (expv[c], 0, 0)),
            pl.BlockSpec((1, F, H), lambda c, expv, valid: (expv[c], 0, 0)),
        ],
        out_specs=pl.BlockSpec((CHUNK, H), lambda c, expv, valid: (c, 0)),
    )
    return pl.pallas_call(
        _ffn_body,
        grid_spec=grid_spec,
        out_shape=jax.ShapeDtypeStruct((NSLOT, H), jnp.float32),
        compiler_params=pltpu.CompilerParams(
            dimension_semantics=("arbitrary",),
        ),
    )(expv, valid, xg, w1b, w2b)


def _combine_body(g_ref, p_ref, y_ref):
    y_ref[...] = (p_ref[:, 0:1] * g_ref[:, 0, :]
                  + p_ref[:, 1:2] * g_ref[:, 1, :])


def _combine(g3, probs):
    return pl.pallas_call(
        _combine_body,
        out_shape=jax.ShapeDtypeStruct((T, H), jnp.float32),
    )(g3, probs)


@jax.jit
def kernel(x, Wr, W1, W2):
    pos, probs, expv, valid = _router(x, Wr)
    pos_flat = pos.reshape(NENT)
    expv_s = expv.reshape(NCPAD)[:NCHUNK]
    valid_s = valid.reshape(NCPAD)[:NCHUNK]

    gidx = _build_gidx(pos_flat, jnp.zeros((NSLOT,), jnp.int32))
    xg = _sc_gather(x, gidx, NSLOT)

    w1b = W1.astype(jnp.bfloat16)
    w2b = W2.astype(jnp.bfloat16)
    o_cmp = _ffn(expv_s, valid_s, xg, w1b, w2b)

    g = _sc_gather(o_cmp, pos_flat, NENT)
    return _combine(g.reshape(T, K, H), probs)
